# baseline (speedup 1.0000x reference)
import functools

import jax
import jax.numpy as jnp
from jax.experimental import pallas as pl

K_NB = 8
EPS = 1e-5
F32 = jnp.float32
BF16 = jnp.bfloat16
HI = jax.lax.Precision.HIGHEST


def _bdot(a, b):
    return jnp.dot(a, b, preferred_element_type=F32)


def _proj_kernel(x_ref, w_ref, o_ref):
    o_ref[...] = _bdot(x_ref[...], w_ref[...])


def _proj_matmul(x, w_bf, rows_per_blk):
    m, d = x.shape
    p = w_bf.shape[1]
    grid = m // rows_per_blk
    return pl.pallas_call(
        _proj_kernel,
        grid=(grid,),
        in_specs=[
            pl.BlockSpec((rows_per_blk, d), lambda i: (i, 0)),
            pl.BlockSpec((d, p), lambda i: (0, 0)),
        ],
        out_specs=pl.BlockSpec((rows_per_blk, p), lambda i: (i, 0)),
        out_shape=jax.ShapeDtypeStruct((m, p), F32),
    )(x, w_bf)


def _knn_kernel(px_ref, py_ref, adj_ref, *, n, npad, k):
    px = px_ref[...]
    py = py_ref[...]
    g = px.shape[0]
    dx = px[:, :, None] - px[:, None, :]
    dy = py[:, :, None] - py[:, None, :]
    dist = dx * dx + dy * dy
    jota = jax.lax.broadcasted_iota(jnp.int32, (g, npad, npad), 2)
    iota_i = jax.lax.broadcasted_iota(jnp.int32, (g, npad, npad), 1)
    enc = jax.lax.bitcast_convert_type(dist, jnp.int32)
    enc = (enc & jnp.int32(~31)) | jota
    bigi = jnp.int32(0x7F000000)
    enc = jnp.where(jota == iota_i, bigi, enc)
    enc = jnp.where(jota >= n, bigi, enc)
    row_pad = jax.lax.broadcasted_iota(jnp.int32, (g, npad, 1), 1) >= n
    for kk in range(k):
        mn = jnp.min(enc, axis=-1, keepdims=True)
        am = jnp.where(row_pad, n, mn & 31)
        adj_ref[:, kk] = am
        enc = jnp.where(jota == am, bigi, enc)


def _knn(px, py, n, npad, k, g_blk):
    bt = px.shape[0]
    grid = bt // g_blk
    return pl.pallas_call(
        functools.partial(_knn_kernel, n=n, npad=npad, k=k),
        grid=(grid,),
        in_specs=[
            pl.BlockSpec((g_blk, npad), lambda i: (i, 0)),
            pl.BlockSpec((g_blk, npad), lambda i: (i, 0)),
        ],
        out_specs=pl.BlockSpec((g_blk, k, npad, 1), lambda i: (i, 0, 0, 0)),
        out_shape=jax.ShapeDtypeStruct((bt, k, npad, 1), jnp.int32),
    )(px, py)


def _norm_scale_shift(s, ss, gamma, beta, cnt):
    mu = s / cnt
    var = ss / cnt - mu * mu
    rs = jax.lax.rsqrt(var + EPS)
    scale = rs * gamma
    shift = beta - mu * scale
    return scale, shift


def _node_mask(rows, npad, n):
    return (jax.lax.broadcasted_iota(jnp.int32, (rows, 1), 0) % npad) < n


def _edge_pre(h, adj4, w1a_bf, w1b_bf, npad, k):
    c, f = h.shape
    g = c // npad
    r = g * k * npad
    fo = w1a_bf.shape[1]
    gidx = jax.lax.broadcasted_iota(jnp.int32, (g, k, npad, 1), 0)
    colidx = (adj4 + gidx * npad).reshape(r, 1)
    jota = jax.lax.broadcasted_iota(jnp.int32, (r, c), 1)
    oh = jnp.where(jota == colidx, 1.0, 0.0)
    h_hi = h.astype(BF16).astype(F32)
    h_lo = h - h_hi
    hn = _bdot(oh, h_hi) + _bdot(oh, h_lo)
    a2d = _bdot(h_hi, w1a_bf)
    ctr = jnp.broadcast_to(h.reshape(g, 1, npad, f),
                           (g, k, npad, f)).reshape(r, f)
    xb = _bdot(hn - ctr, w1b_bf)
    a_rep = jnp.broadcast_to(a2d.reshape(g, 1, npad, fo),
                             (g, k, npad, fo)).reshape(r, fo)
    return a_rep + xb


def _accum_stats(pid, s_ref, ss_ref, s, ss):
    @pl.when(pid == 0)
    def _():
        s_ref[...] = s
        ss_ref[...] = ss

    @pl.when(pid != 0)
    def _():
        s_ref[...] += s
        ss_ref[...] += ss


def _prep_h(h_ref, prev, n, npad, cnt):
    if prev is None:
        h = h_ref[...]
    else:
        ps_ref, pss_ref, pg_ref, pb_ref = prev
        scale, shift = _norm_scale_shift(ps_ref[...], pss_ref[...],
                                         pg_ref[...], pb_ref[...], cnt)
        h = jnp.maximum(h_ref[...] * scale + shift, 0.0)
    return jnp.where(_node_mask(h.shape[0], npad, n), h, 0.0)


def _p1_kernel(*refs, n, npad, k, has_prev, cnt):
    if has_prev:
        (h_ref, wa_ref, wb_ref, adj_ref, ps_ref, pss_ref, pg_ref, pb_ref,
         xe_out, s_out, ss_out) = refs
        prev = (ps_ref, pss_ref, pg_ref, pb_ref)
    else:
        h_ref, wa_ref, wb_ref, adj_ref, xe_out, s_out, ss_out = refs
        prev = None
    pid = pl.program_id(0)
    h = _prep_h(h_ref, prev, n, npad, cnt)
    xe = _edge_pre(h, adj_ref[...], wa_ref[...], wb_ref[...], npad, k)
    xe_out[...] = xe
    s = jnp.sum(xe, axis=0)[None, :]
    ss = jnp.sum(xe * xe, axis=0)[None, :]
    _accum_stats(pid, s_out, ss_out, s, ss)


def _p2_kernel(*refs, n, npad, k, has_prev, cnt):
    (xe_ref, s1_ref, ss1_ref, g1_ref, b1_ref, w2_ref,
     ym_out, s_out, ss_out) = refs
    pid = pl.program_id(0)
    xe = xe_ref[...]
    r = xe.shape[0]
    g = r // (k * npad)
    gn = g * npad
    fo = w2_ref.shape[1]
    scale, shift = _norm_scale_shift(s1_ref[...], ss1_ref[...],
                                     g1_ref[...], b1_ref[...], cnt)
    z = jnp.maximum(xe * scale + shift, 0.0)
    y = _bdot(z, w2_ref[...])
    y_pad = _bdot(jnp.maximum(shift, 0.0), w2_ref[...])
    n_pad = float(g * (npad - n) * k)
    s = jnp.sum(y, axis=0)[None, :] - n_pad * y_pad
    ss = jnp.sum(y * y, axis=0)[None, :] - n_pad * (y_pad * y_pad)
    _accum_stats(pid, s_out, ss_out, s, ss)
    y4 = y.reshape(g, k, npad, fo)
    ym = y4[:, 0]
    for kk in range(1, k):
        ym = jnp.maximum(ym, y4[:, kk])
    ym_out[...] = ym.reshape(gn, fo)


def _full_spec(shp):
    return pl.BlockSpec(shp, lambda i: tuple(0 for _ in shp))


def _run_p1(h, wa, wb, adj, n, npad, k, g_blk, prev_stats, cnt):
    bt = adj.shape[0]
    grid = bt // g_blk
    rows = g_blk * npad
    m, f = h.shape
    fo = wa.shape[1]
    in_specs = [
        pl.BlockSpec((rows, f), lambda i: (i, 0)),
        _full_spec(wa.shape),
        _full_spec(wb.shape),
        pl.BlockSpec((g_blk, k, npad, 1), lambda i: (i, 0, 0, 0)),
    ]
    args = [h, wa, wb, adj]
    has_prev = prev_stats is not None
    if has_prev:
        pf = prev_stats[0].shape[1]
        in_specs += [_full_spec((1, pf))] * 4
        args += list(prev_stats)
    r_blk = g_blk * k * npad
    return pl.pallas_call(
        functools.partial(_p1_kernel, n=n, npad=npad, k=k,
                          has_prev=has_prev, cnt=cnt),
        grid=(grid,),
        in_specs=in_specs,
        out_specs=(
            pl.BlockSpec((r_blk, fo), lambda i: (i, 0)),
            pl.BlockSpec((1, fo), lambda i: (0, 0)),
            pl.BlockSpec((1, fo), lambda i: (0, 0)),
        ),
        out_shape=(
            jax.ShapeDtypeStruct((bt * k * npad, fo), F32),
            jax.ShapeDtypeStruct((1, fo), F32),
            jax.ShapeDtypeStruct((1, fo), F32),
        ),
    )(*args)


def _run_p2(xe, s1, ss1, g1, b1, w2t, bt, n, npad, k, g_blk, cnt):
    grid = bt // g_blk
    rows = g_blk * npad
    r_blk = g_blk * k * npad
    m = bt * npad
    f1 = xe.shape[1]
    fo = w2t.shape[1]
    in_specs = [
        pl.BlockSpec((r_blk, f1), lambda i: (i, 0)),
        _full_spec((1, f1)),
        _full_spec((1, f1)),
        _full_spec((1, f1)),
        _full_spec((1, f1)),
        _full_spec((f1, fo)),
    ]
    args = [xe, s1, ss1, g1, b1, w2t]
    return pl.pallas_call(
        functools.partial(_p2_kernel, n=n, npad=npad, k=k,
                          has_prev=False, cnt=cnt),
        grid=(grid,),
        in_specs=in_specs,
        out_specs=(
            pl.BlockSpec((rows, fo), lambda i: (i, 0)),
            pl.BlockSpec((1, fo), lambda i: (0, 0)),
            pl.BlockSpec((1, fo), lambda i: (0, 0)),
        ),
        out_shape=(
            jax.ShapeDtypeStruct((m, fo), F32),
            jax.ShapeDtypeStruct((1, fo), F32),
            jax.ShapeDtypeStruct((1, fo), F32),
        ),
    )(*args)


def _final_kernel(y_ref, s_ref, ss_ref, g_ref, b_ref, o_ref, *, cnt):
    scale, shift = _norm_scale_shift(s_ref[...], ss_ref[...],
                                     g_ref[...], b_ref[...], cnt)
    o_ref[...] = jnp.maximum(y_ref[...] * scale + shift, 0.0)


def _run_final(y, s, ss, g2, b2, rows_per_blk, cnt):
    m, f = y.shape
    grid = m // rows_per_blk
    return pl.pallas_call(
        functools.partial(_final_kernel, cnt=cnt),
        grid=(grid,),
        in_specs=[
            pl.BlockSpec((rows_per_blk, f), lambda i: (i, 0)),
            _full_spec((1, f)), _full_spec((1, f)),
            _full_spec((1, f)), _full_spec((1, f)),
        ],
        out_specs=pl.BlockSpec((rows_per_blk, f), lambda i: (i, 0)),
        out_shape=jax.ShapeDtypeStruct((m, f), F32),
    )(y, s, ss, g2, b2)


def kernel(phi_x3d, positions, velocities, team_ids, player_mask, W_proj,
           W1_0, g1_0, b1_0, W2_0, g2_0, b2_0,
           W1_1, g1_1, b1_1, W2_1, g2_1, b2_1,
           W1_2, g1_2, b1_2, W2_2, g2_2, b2_2):
    B, N, T, D = phi_x3d.shape
    P = W_proj.shape[0]
    k = K_NB
    NP = 24
    BT = B * T
    MP = BT * NP
    cnt = float(BT * N * k)

    phi_proj = _proj_matmul(phi_x3d.reshape(B * N * T, D),
                            W_proj.T, 512)

    phi_bt = phi_proj.reshape(B, N, T, P).transpose(0, 2, 1, 3)
    pos_bt = positions.transpose(0, 2, 1, 3)
    vel_bt = velocities.transpose(0, 2, 1, 3)
    team_bt = team_ids.astype(F32).transpose(0, 2, 1)[..., None]
    f_in = 5 + P
    f_pad = 128
    h0 = jnp.concatenate(
        [pos_bt, vel_bt, team_bt, phi_bt,
         jnp.zeros((B, T, N, f_pad - f_in), F32)], axis=-1)
    h0 = jnp.pad(h0, ((0, 0), (0, 0), (0, NP - N), (0, 0)))
    h0 = h0.reshape(MP, f_pad)

    px = jnp.pad(positions[..., 0].transpose(0, 2, 1),
                 ((0, 0), (0, 0), (0, NP - N))).reshape(BT, NP)
    py = jnp.pad(positions[..., 1].transpose(0, 2, 1),
                 ((0, 0), (0, 0), (0, NP - N))).reshape(BT, NP)
    adj = _knn(px, py, N, NP, k, 128)

    layer_params = [
        (W1_0, g1_0, b1_0, W2_0, g2_0, b2_0),
        (W1_1, g1_1, b1_1, W2_1, g2_1, b2_1),
        (W1_2, g1_2, b1_2, W2_2, g2_2, b2_2),
    ]

    g_blk = 32
    h = h0
    prev_stats = None
    for li, (W1, g1, b1, W2, g2, b2) in enumerate(layer_params):
        ind = W1.shape[1] // 2
        f_h = h.shape[1]
        wa = jnp.zeros((f_h, W1.shape[0]), F32).at[:ind].set(
            W1[:, :ind].T)
        wb = jnp.zeros((f_h, W1.shape[0]), F32).at[:ind].set(
            W1[:, ind:].T)
        xe, s1, ss1 = _run_p1(h, wa, wb, adj, N, NP, k, g_blk,
                              prev_stats, cnt)
        ym, s2, ss2 = _run_p2(xe, s1, ss1,
                              g1[None, :], b1[None, :],
                              W2.T,
                              BT, N, NP, k, 2 * g_blk, cnt)
        h = ym
        prev_stats = (s2, ss2, g2[None, :], b2[None, :])

    s2f, ss2f, g2r, b2r = prev_stats
    h_fin = _run_final(h, s2f, ss2f, g2r, b2r, NP * 16, cnt)
    out = h_fin.reshape(B, T, NP, -1)[:, :, :N, :].transpose(0, 2, 1, 3)
    return out


# speedup vs baseline: 1.1394x; 1.1394x over previous
import functools

import jax
import jax.numpy as jnp
from jax.experimental import pallas as pl

K_NB = 8
EPS = 1e-5
F32 = jnp.float32
BF16 = jnp.bfloat16
HI = jax.lax.Precision.HIGHEST


def _bdot(a, b):
    return jnp.dot(a, b, preferred_element_type=F32)


def _proj_kernel(x_ref, w_ref, o_ref):
    o_ref[...] = _bdot(x_ref[...], w_ref[...])


def _proj_matmul(x, w_bf, rows_per_blk):
    m, d = x.shape
    p = w_bf.shape[1]
    grid = m // rows_per_blk
    return pl.pallas_call(
        _proj_kernel,
        grid=(grid,),
        in_specs=[
            pl.BlockSpec((rows_per_blk, d), lambda i: (i, 0)),
            pl.BlockSpec((d, p), lambda i: (0, 0)),
        ],
        out_specs=pl.BlockSpec((rows_per_blk, p), lambda i: (i, 0)),
        out_shape=jax.ShapeDtypeStruct((m, p), F32),
    )(x, w_bf)


def _knn_kernel(px_ref, py_ref, adj_ref, *, n, npad, k):
    px = px_ref[...]
    py = py_ref[...]
    g = px.shape[0]
    dx = px[:, :, None] - px[:, None, :]
    dy = py[:, :, None] - py[:, None, :]
    dist = dx * dx + dy * dy
    jota = jax.lax.broadcasted_iota(jnp.int32, (g, npad, npad), 2)
    iota_i = jax.lax.broadcasted_iota(jnp.int32, (g, npad, npad), 1)
    enc = jax.lax.bitcast_convert_type(dist, jnp.int32)
    enc = (enc & jnp.int32(~31)) | jota
    bigi = jnp.int32(0x7F000000)
    enc = jnp.where(jota == iota_i, bigi, enc)
    enc = jnp.where(jota >= n, bigi, enc)
    row_pad = jax.lax.broadcasted_iota(jnp.int32, (g, npad, 1), 1) >= n
    for kk in range(k):
        mn = jnp.min(enc, axis=-1, keepdims=True)
        am = jnp.where(row_pad, n, mn & 31)
        adj_ref[:, kk] = am
        enc = jnp.where(jota == am, bigi, enc)


def _knn(px, py, n, npad, k, g_blk):
    bt = px.shape[0]
    grid = bt // g_blk
    return pl.pallas_call(
        functools.partial(_knn_kernel, n=n, npad=npad, k=k),
        grid=(grid,),
        in_specs=[
            pl.BlockSpec((g_blk, npad), lambda i: (i, 0)),
            pl.BlockSpec((g_blk, npad), lambda i: (i, 0)),
        ],
        out_specs=pl.BlockSpec((g_blk, k, npad, 1), lambda i: (i, 0, 0, 0)),
        out_shape=jax.ShapeDtypeStruct((bt, k, npad, 1), jnp.int32),
    )(px, py)


def _norm_scale_shift(s, ss, gamma, beta, cnt):
    mu = s / cnt
    var = ss / cnt - mu * mu
    rs = jax.lax.rsqrt(var + EPS)
    scale = rs * gamma
    shift = beta - mu * scale
    return scale, shift


def _node_mask(rows, npad, n):
    return (jax.lax.broadcasted_iota(jnp.int32, (rows, 1), 0) % npad) < n


def _edge_pre(h, adj4, w1a_bf, w1b_bf, npad, k):
    c, f = h.shape
    g = c // npad
    r = g * k * npad
    fo = w1a_bf.shape[1]
    gidx = jax.lax.broadcasted_iota(jnp.int32, (g, k, npad, 1), 0)
    colidx = (adj4 + gidx * npad).reshape(r, 1)
    jota = jax.lax.broadcasted_iota(jnp.int32, (r, c), 1)
    oh = jnp.where(jota == colidx, 1.0, 0.0)
    h_hi = h.astype(BF16).astype(F32)
    h_lo = h - h_hi
    hn = _bdot(oh, h_hi) + _bdot(oh, h_lo)
    a2d = _bdot(h_hi, w1a_bf)
    ctr = jnp.broadcast_to(h.reshape(g, 1, npad, f),
                           (g, k, npad, f)).reshape(r, f)
    xb = _bdot(hn - ctr, w1b_bf)
    a_rep = jnp.broadcast_to(a2d.reshape(g, 1, npad, fo),
                             (g, k, npad, fo)).reshape(r, fo)
    return a_rep + xb


def _accum_stats(pid, s_ref, ss_ref, s, ss):
    @pl.when(pid == 0)
    def _():
        s_ref[...] = s
        ss_ref[...] = ss

    @pl.when(pid != 0)
    def _():
        s_ref[...] += s
        ss_ref[...] += ss


def _prep_h(h_ref, prev, n, npad, cnt):
    if prev is None:
        h = h_ref[...]
    else:
        ps_ref, pss_ref, pg_ref, pb_ref = prev
        scale, shift = _norm_scale_shift(ps_ref[...], pss_ref[...],
                                         pg_ref[...], pb_ref[...], cnt)
        h = jnp.maximum(h_ref[...] * scale + shift, 0.0)
    return jnp.where(_node_mask(h.shape[0], npad, n), h, 0.0)


def _p1_kernel(*refs, n, npad, k, has_prev, cnt):
    if has_prev:
        (h_ref, wa_ref, wb_ref, adj_ref, ps_ref, pss_ref, pg_ref, pb_ref,
         xe_out, s_out, ss_out) = refs
        prev = (ps_ref, pss_ref, pg_ref, pb_ref)
    else:
        h_ref, wa_ref, wb_ref, adj_ref, xe_out, s_out, ss_out = refs
        prev = None
    pid = pl.program_id(0)
    h = _prep_h(h_ref, prev, n, npad, cnt)
    xe = _edge_pre(h, adj_ref[...], wa_ref[...], wb_ref[...], npad, k)
    xe_out[...] = xe
    s = jnp.sum(xe, axis=0)[None, :]
    ss = jnp.sum(xe * xe, axis=0)[None, :]
    _accum_stats(pid, s_out, ss_out, s, ss)


def _p2_kernel(*refs, n, npad, k, has_prev, cnt):
    (xe_ref, s1_ref, ss1_ref, g1_ref, b1_ref, w2_ref,
     ym_out, s_out, ss_out) = refs
    pid = pl.program_id(0)
    xe = xe_ref[...]
    r = xe.shape[0]
    g = r // (k * npad)
    gn = g * npad
    fo = w2_ref.shape[1]
    scale, shift = _norm_scale_shift(s1_ref[...], ss1_ref[...],
                                     g1_ref[...], b1_ref[...], cnt)
    z = jnp.maximum(xe * scale + shift, 0.0)
    y = _bdot(z, w2_ref[...])
    y_pad = _bdot(jnp.maximum(shift, 0.0), w2_ref[...])
    n_pad = float(g * (npad - n) * k)
    s = jnp.sum(y, axis=0)[None, :] - n_pad * y_pad
    ss = jnp.sum(y * y, axis=0)[None, :] - n_pad * (y_pad * y_pad)
    _accum_stats(pid, s_out, ss_out, s, ss)
    y4 = y.reshape(g, k, npad, fo)
    ym = y4[:, 0]
    for kk in range(1, k):
        ym = jnp.maximum(ym, y4[:, kk])
    ym_out[...] = ym.reshape(gn, fo)


def _full_spec(shp):
    return pl.BlockSpec(shp, lambda i: tuple(0 for _ in shp))


def _run_p1(h, wa, wb, adj, n, npad, k, g_blk, prev_stats, cnt):
    bt = adj.shape[0]
    grid = bt // g_blk
    rows = g_blk * npad
    m, f = h.shape
    fo = wa.shape[1]
    in_specs = [
        pl.BlockSpec((rows, f), lambda i: (i, 0)),
        _full_spec(wa.shape),
        _full_spec(wb.shape),
        pl.BlockSpec((g_blk, k, npad, 1), lambda i: (i, 0, 0, 0)),
    ]
    args = [h, wa, wb, adj]
    has_prev = prev_stats is not None
    if has_prev:
        pf = prev_stats[0].shape[1]
        in_specs += [_full_spec((1, pf))] * 4
        args += list(prev_stats)
    r_blk = g_blk * k * npad
    return pl.pallas_call(
        functools.partial(_p1_kernel, n=n, npad=npad, k=k,
                          has_prev=has_prev, cnt=cnt),
        grid=(grid,),
        in_specs=in_specs,
        out_specs=(
            pl.BlockSpec((r_blk, fo), lambda i: (i, 0)),
            pl.BlockSpec((1, fo), lambda i: (0, 0)),
            pl.BlockSpec((1, fo), lambda i: (0, 0)),
        ),
        out_shape=(
            jax.ShapeDtypeStruct((bt * k * npad, fo), F32),
            jax.ShapeDtypeStruct((1, fo), F32),
            jax.ShapeDtypeStruct((1, fo), F32),
        ),
    )(*args)


def _run_p2(xe, s1, ss1, g1, b1, w2t, bt, n, npad, k, g_blk, cnt):
    grid = bt // g_blk
    rows = g_blk * npad
    r_blk = g_blk * k * npad
    m = bt * npad
    f1 = xe.shape[1]
    fo = w2t.shape[1]
    in_specs = [
        pl.BlockSpec((r_blk, f1), lambda i: (i, 0)),
        _full_spec((1, f1)),
        _full_spec((1, f1)),
        _full_spec((1, f1)),
        _full_spec((1, f1)),
        _full_spec((f1, fo)),
    ]
    args = [xe, s1, ss1, g1, b1, w2t]
    return pl.pallas_call(
        functools.partial(_p2_kernel, n=n, npad=npad, k=k,
                          has_prev=False, cnt=cnt),
        grid=(grid,),
        in_specs=in_specs,
        out_specs=(
            pl.BlockSpec((rows, fo), lambda i: (i, 0)),
            pl.BlockSpec((1, fo), lambda i: (0, 0)),
            pl.BlockSpec((1, fo), lambda i: (0, 0)),
        ),
        out_shape=(
            jax.ShapeDtypeStruct((m, fo), F32),
            jax.ShapeDtypeStruct((1, fo), F32),
            jax.ShapeDtypeStruct((1, fo), F32),
        ),
    )(*args)


def _final_kernel(y_ref, s_ref, ss_ref, g_ref, b_ref, o_ref, *, cnt):
    scale, shift = _norm_scale_shift(s_ref[...], ss_ref[...],
                                     g_ref[...], b_ref[...], cnt)
    o_ref[...] = jnp.maximum(y_ref[...] * scale + shift, 0.0)


def _run_final(y, s, ss, g2, b2, rows_per_blk, cnt):
    m, f = y.shape
    grid = m // rows_per_blk
    return pl.pallas_call(
        functools.partial(_final_kernel, cnt=cnt),
        grid=(grid,),
        in_specs=[
            pl.BlockSpec((rows_per_blk, f), lambda i: (i, 0)),
            _full_spec((1, f)), _full_spec((1, f)),
            _full_spec((1, f)), _full_spec((1, f)),
        ],
        out_specs=pl.BlockSpec((rows_per_blk, f), lambda i: (i, 0)),
        out_shape=jax.ShapeDtypeStruct((m, f), F32),
    )(y, s, ss, g2, b2)


def kernel(phi_x3d, positions, velocities, team_ids, player_mask, W_proj,
           W1_0, g1_0, b1_0, W2_0, g2_0, b2_0,
           W1_1, g1_1, b1_1, W2_1, g2_1, b2_1,
           W1_2, g1_2, b1_2, W2_2, g2_2, b2_2):
    B, N, T, D = phi_x3d.shape
    P = W_proj.shape[0]
    k = K_NB
    NP = 24
    BT = B * T
    MP = BT * NP
    cnt = float(BT * N * k)

    phi_proj = _proj_matmul(phi_x3d.reshape(B * N * T, D),
                            W_proj.T, 512)

    phi_bt = phi_proj.reshape(B, N, T, P).transpose(0, 2, 1, 3)
    pos_bt = positions.transpose(0, 2, 1, 3)
    vel_bt = velocities.transpose(0, 2, 1, 3)
    team_bt = team_ids.astype(F32).transpose(0, 2, 1)[..., None]
    f_in = 5 + P
    f_pad = 128
    h0 = jnp.concatenate(
        [pos_bt, vel_bt, team_bt, phi_bt,
         jnp.zeros((B, T, N, f_pad - f_in), F32)], axis=-1)
    h0 = jnp.pad(h0, ((0, 0), (0, 0), (0, NP - N), (0, 0)))
    h0 = h0.reshape(MP, f_pad)

    px = jnp.pad(positions[..., 0].transpose(0, 2, 1),
                 ((0, 0), (0, 0), (0, NP - N))).reshape(BT, NP)
    py = jnp.pad(positions[..., 1].transpose(0, 2, 1),
                 ((0, 0), (0, 0), (0, NP - N))).reshape(BT, NP)
    adj = _knn(px, py, N, NP, k, 128)

    layer_params = [
        (W1_0, g1_0, b1_0, W2_0, g2_0, b2_0),
        (W1_1, g1_1, b1_1, W2_1, g2_1, b2_1),
        (W1_2, g1_2, b1_2, W2_2, g2_2, b2_2),
    ]

    g_blk = 16
    h = h0
    prev_stats = None
    for li, (W1, g1, b1, W2, g2, b2) in enumerate(layer_params):
        ind = W1.shape[1] // 2
        f_h = h.shape[1]
        wa = jnp.zeros((f_h, W1.shape[0]), F32).at[:ind].set(
            W1[:, :ind].T)
        wb = jnp.zeros((f_h, W1.shape[0]), F32).at[:ind].set(
            W1[:, ind:].T)
        xe, s1, ss1 = _run_p1(h, wa, wb, adj, N, NP, k, g_blk,
                              prev_stats, cnt)
        ym, s2, ss2 = _run_p2(xe, s1, ss1,
                              g1[None, :], b1[None, :],
                              W2.T,
                              BT, N, NP, k, 2 * g_blk, cnt)
        h = ym
        prev_stats = (s2, ss2, g2[None, :], b2[None, :])

    s2f, ss2f, g2r, b2r = prev_stats
    h_fin = _run_final(h, s2f, ss2f, g2r, b2r, NP * 16, cnt)
    out = h_fin.reshape(B, T, NP, -1)[:, :, :N, :].transpose(0, 2, 1, 3)
    return out


# speedup vs baseline: 1.1756x; 1.0317x over previous
import functools

import jax
import jax.numpy as jnp
from jax.experimental import pallas as pl

K_NB = 8
EPS = 1e-5
F32 = jnp.float32
BF16 = jnp.bfloat16
HI = jax.lax.Precision.HIGHEST


def _bdot(a, b):
    return jnp.dot(a, b, preferred_element_type=F32)


def _proj_kernel(x_ref, w_ref, o_ref):
    o_ref[...] = _bdot(x_ref[...], w_ref[...])


def _proj_matmul(x, w_bf, rows_per_blk):
    m, d = x.shape
    p = w_bf.shape[1]
    grid = m // rows_per_blk
    return pl.pallas_call(
        _proj_kernel,
        grid=(grid,),
        in_specs=[
            pl.BlockSpec((rows_per_blk, d), lambda i: (i, 0)),
            pl.BlockSpec((d, p), lambda i: (0, 0)),
        ],
        out_specs=pl.BlockSpec((rows_per_blk, p), lambda i: (i, 0)),
        out_shape=jax.ShapeDtypeStruct((m, p), F32),
    )(x, w_bf)


def _knn_kernel(px_ref, py_ref, adj_ref, *, n, npad, k):
    px = px_ref[...]
    py = py_ref[...]
    g = px.shape[0]
    dx = px[:, :, None] - px[:, None, :]
    dy = py[:, :, None] - py[:, None, :]
    dist = dx * dx + dy * dy
    jota = jax.lax.broadcasted_iota(jnp.int32, (g, npad, npad), 2)
    iota_i = jax.lax.broadcasted_iota(jnp.int32, (g, npad, npad), 1)
    enc = jax.lax.bitcast_convert_type(dist, jnp.int32)
    enc = (enc & jnp.int32(~31)) | jota
    bigi = jnp.int32(0x7F000000)
    enc = jnp.where(jota == iota_i, bigi, enc)
    enc = jnp.where(jota >= n, bigi, enc)
    row_pad = jax.lax.broadcasted_iota(jnp.int32, (g, npad, 1), 1) >= n
    for kk in range(k):
        mn = jnp.min(enc, axis=-1, keepdims=True)
        am = jnp.where(row_pad, n, mn & 31)
        adj_ref[:, kk] = am
        enc = jnp.where(jota == am, bigi, enc)


def _knn(px, py, n, npad, k, g_blk):
    bt = px.shape[0]
    grid = bt // g_blk
    return pl.pallas_call(
        functools.partial(_knn_kernel, n=n, npad=npad, k=k),
        grid=(grid,),
        in_specs=[
            pl.BlockSpec((g_blk, npad), lambda i: (i, 0)),
            pl.BlockSpec((g_blk, npad), lambda i: (i, 0)),
        ],
        out_specs=pl.BlockSpec((g_blk, k, npad, 1), lambda i: (i, 0, 0, 0)),
        out_shape=jax.ShapeDtypeStruct((bt, k, npad, 1), jnp.int32),
    )(px, py)


def _norm_scale_shift(s, ss, gamma, beta, cnt):
    mu = s / cnt
    var = ss / cnt - mu * mu
    rs = jax.lax.rsqrt(var + EPS)
    scale = rs * gamma
    shift = beta - mu * scale
    return scale, shift


def _node_mask(rows, npad, n):
    return (jax.lax.broadcasted_iota(jnp.int32, (rows, 1), 0) % npad) < n


def _edge_pre(h, adj4, w1a_bf, w1b_bf, npad, k):
    c, f = h.shape
    g = c // npad
    r = g * k * npad
    fo = w1a_bf.shape[1]
    gidx = jax.lax.broadcasted_iota(jnp.int32, (g, k, npad, 1), 0)
    colidx = (adj4 + gidx * npad).reshape(r, 1)
    jota = jax.lax.broadcasted_iota(jnp.int32, (r, c), 1)
    oh = jnp.where(jota == colidx, 1.0, 0.0)
    h_hi = h.astype(BF16).astype(F32)
    h_lo = h - h_hi
    hn = _bdot(oh, h_hi) + _bdot(oh, h_lo)
    a2d = _bdot(h_hi, w1a_bf)
    ctr = jnp.broadcast_to(h.reshape(g, 1, npad, f),
                           (g, k, npad, f)).reshape(r, f)
    xb = _bdot(hn - ctr, w1b_bf)
    a_rep = jnp.broadcast_to(a2d.reshape(g, 1, npad, fo),
                             (g, k, npad, fo)).reshape(r, fo)
    return a_rep + xb


def _accum_stats(pid, s_ref, ss_ref, s, ss):
    @pl.when(pid == 0)
    def _():
        s_ref[...] = s
        ss_ref[...] = ss

    @pl.when(pid != 0)
    def _():
        s_ref[...] += s
        ss_ref[...] += ss


def _prep_h(h_ref, prev, n, npad, cnt):
    if prev is None:
        h = h_ref[...]
    else:
        ps_ref, pss_ref, pg_ref, pb_ref = prev
        scale, shift = _norm_scale_shift(ps_ref[...], pss_ref[...],
                                         pg_ref[...], pb_ref[...], cnt)
        h = jnp.maximum(h_ref[...] * scale + shift, 0.0)
    return jnp.where(_node_mask(h.shape[0], npad, n), h, 0.0)


def _p1_kernel(*refs, n, npad, k, has_prev, cnt):
    if has_prev:
        (h_ref, wa_ref, wb_ref, adj_ref, ps_ref, pss_ref, pg_ref, pb_ref,
         xe_out, s_out, ss_out) = refs
        prev = (ps_ref, pss_ref, pg_ref, pb_ref)
    else:
        h_ref, wa_ref, wb_ref, adj_ref, xe_out, s_out, ss_out = refs
        prev = None
    pid = pl.program_id(0)
    h = _prep_h(h_ref, prev, n, npad, cnt)
    xe = _edge_pre(h, adj_ref[...], wa_ref[...], wb_ref[...], npad, k)
    xe_out[...] = xe
    s = jnp.sum(xe, axis=0)[None, :]
    ss = jnp.sum(xe * xe, axis=0)[None, :]
    _accum_stats(pid, s_out, ss_out, s, ss)


def _p2_kernel(*refs, n, npad, k, has_prev, cnt):
    (xe_ref, s1_ref, ss1_ref, g1_ref, b1_ref, w2_ref,
     ym_out, s_out, ss_out) = refs
    pid = pl.program_id(0)
    xe = xe_ref[...]
    r = xe.shape[0]
    g = r // (k * npad)
    gn = g * npad
    fo = w2_ref.shape[1]
    scale, shift = _norm_scale_shift(s1_ref[...], ss1_ref[...],
                                     g1_ref[...], b1_ref[...], cnt)
    z = jnp.maximum(xe * scale + shift, 0.0)
    y = _bdot(z, w2_ref[...])
    y_pad = _bdot(jnp.maximum(shift, 0.0), w2_ref[...])
    n_pad = float(g * (npad - n) * k)
    s = jnp.sum(y, axis=0)[None, :] - n_pad * y_pad
    ss = jnp.sum(y * y, axis=0)[None, :] - n_pad * (y_pad * y_pad)
    _accum_stats(pid, s_out, ss_out, s, ss)
    y4 = y.reshape(g, k, npad, fo)
    ym = y4[:, 0]
    for kk in range(1, k):
        ym = jnp.maximum(ym, y4[:, kk])
    ym_out[...] = ym.reshape(gn, fo)


def _full_spec(shp):
    return pl.BlockSpec(shp, lambda i: tuple(0 for _ in shp))


def _run_p1(h, wa, wb, adj, n, npad, k, g_blk, prev_stats, cnt):
    bt = adj.shape[0]
    grid = bt // g_blk
    rows = g_blk * npad
    m, f = h.shape
    fo = wa.shape[1]
    in_specs = [
        pl.BlockSpec((rows, f), lambda i: (i, 0)),
        _full_spec(wa.shape),
        _full_spec(wb.shape),
        pl.BlockSpec((g_blk, k, npad, 1), lambda i: (i, 0, 0, 0)),
    ]
    args = [h, wa, wb, adj]
    has_prev = prev_stats is not None
    if has_prev:
        pf = prev_stats[0].shape[1]
        in_specs += [_full_spec((1, pf))] * 4
        args += list(prev_stats)
    r_blk = g_blk * k * npad
    return pl.pallas_call(
        functools.partial(_p1_kernel, n=n, npad=npad, k=k,
                          has_prev=has_prev, cnt=cnt),
        grid=(grid,),
        in_specs=in_specs,
        out_specs=(
            pl.BlockSpec((r_blk, fo), lambda i: (i, 0)),
            pl.BlockSpec((1, fo), lambda i: (0, 0)),
            pl.BlockSpec((1, fo), lambda i: (0, 0)),
        ),
        out_shape=(
            jax.ShapeDtypeStruct((bt * k * npad, fo), F32),
            jax.ShapeDtypeStruct((1, fo), F32),
            jax.ShapeDtypeStruct((1, fo), F32),
        ),
    )(*args)


def _run_p2(xe, s1, ss1, g1, b1, w2t, bt, n, npad, k, g_blk, cnt):
    grid = bt // g_blk
    rows = g_blk * npad
    r_blk = g_blk * k * npad
    m = bt * npad
    f1 = xe.shape[1]
    fo = w2t.shape[1]
    in_specs = [
        pl.BlockSpec((r_blk, f1), lambda i: (i, 0)),
        _full_spec((1, f1)),
        _full_spec((1, f1)),
        _full_spec((1, f1)),
        _full_spec((1, f1)),
        _full_spec((f1, fo)),
    ]
    args = [xe, s1, ss1, g1, b1, w2t]
    return pl.pallas_call(
        functools.partial(_p2_kernel, n=n, npad=npad, k=k,
                          has_prev=False, cnt=cnt),
        grid=(grid,),
        in_specs=in_specs,
        out_specs=(
            pl.BlockSpec((rows, fo), lambda i: (i, 0)),
            pl.BlockSpec((1, fo), lambda i: (0, 0)),
            pl.BlockSpec((1, fo), lambda i: (0, 0)),
        ),
        out_shape=(
            jax.ShapeDtypeStruct((m, fo), F32),
            jax.ShapeDtypeStruct((1, fo), F32),
            jax.ShapeDtypeStruct((1, fo), F32),
        ),
    )(*args)


def _final_kernel(y_ref, s_ref, ss_ref, g_ref, b_ref, o_ref, *, cnt):
    scale, shift = _norm_scale_shift(s_ref[...], ss_ref[...],
                                     g_ref[...], b_ref[...], cnt)
    o_ref[...] = jnp.maximum(y_ref[...] * scale + shift, 0.0)


def _run_final(y, s, ss, g2, b2, rows_per_blk, cnt):
    m, f = y.shape
    grid = m // rows_per_blk
    return pl.pallas_call(
        functools.partial(_final_kernel, cnt=cnt),
        grid=(grid,),
        in_specs=[
            pl.BlockSpec((rows_per_blk, f), lambda i: (i, 0)),
            _full_spec((1, f)), _full_spec((1, f)),
            _full_spec((1, f)), _full_spec((1, f)),
        ],
        out_specs=pl.BlockSpec((rows_per_blk, f), lambda i: (i, 0)),
        out_shape=jax.ShapeDtypeStruct((m, f), F32),
    )(y, s, ss, g2, b2)


def kernel(phi_x3d, positions, velocities, team_ids, player_mask, W_proj,
           W1_0, g1_0, b1_0, W2_0, g2_0, b2_0,
           W1_1, g1_1, b1_1, W2_1, g2_1, b2_1,
           W1_2, g1_2, b1_2, W2_2, g2_2, b2_2):
    B, N, T, D = phi_x3d.shape
    P = W_proj.shape[0]
    k = K_NB
    NP = 24
    BT = B * T
    MP = BT * NP
    cnt = float(BT * N * k)

    phi_proj = _proj_matmul(phi_x3d.reshape(B * N * T, D),
                            W_proj.T, 1024)

    phi_bt = phi_proj.reshape(B, N, T, P).transpose(0, 2, 1, 3)
    pos_bt = positions.transpose(0, 2, 1, 3)
    vel_bt = velocities.transpose(0, 2, 1, 3)
    team_bt = team_ids.astype(F32).transpose(0, 2, 1)[..., None]
    f_in = 5 + P
    f_pad = 128
    h0 = jnp.concatenate(
        [pos_bt, vel_bt, team_bt, phi_bt,
         jnp.zeros((B, T, N, f_pad - f_in), F32)], axis=-1)
    h0 = jnp.pad(h0, ((0, 0), (0, 0), (0, NP - N), (0, 0)))
    h0 = h0.reshape(MP, f_pad)

    px = jnp.pad(positions[..., 0].transpose(0, 2, 1),
                 ((0, 0), (0, 0), (0, NP - N))).reshape(BT, NP)
    py = jnp.pad(positions[..., 1].transpose(0, 2, 1),
                 ((0, 0), (0, 0), (0, NP - N))).reshape(BT, NP)
    adj = _knn(px, py, N, NP, k, 256)

    layer_params = [
        (W1_0, g1_0, b1_0, W2_0, g2_0, b2_0),
        (W1_1, g1_1, b1_1, W2_1, g2_1, b2_1),
        (W1_2, g1_2, b1_2, W2_2, g2_2, b2_2),
    ]

    g_blk = 16
    h = h0
    prev_stats = None
    for li, (W1, g1, b1, W2, g2, b2) in enumerate(layer_params):
        ind = W1.shape[1] // 2
        f_h = h.shape[1]
        wa = jnp.zeros((f_h, W1.shape[0]), F32).at[:ind].set(
            W1[:, :ind].T)
        wb = jnp.zeros((f_h, W1.shape[0]), F32).at[:ind].set(
            W1[:, ind:].T)
        xe, s1, ss1 = _run_p1(h, wa, wb, adj, N, NP, k, g_blk,
                              prev_stats, cnt)
        ym, s2, ss2 = _run_p2(xe, s1, ss1,
                              g1[None, :], b1[None, :],
                              W2.T,
                              BT, N, NP, k, 2 * g_blk, cnt)
        h = ym
        prev_stats = (s2, ss2, g2[None, :], b2[None, :])

    s2f, ss2f, g2r, b2r = prev_stats
    h_fin = _run_final(h, s2f, ss2f, g2r, b2r, NP * 64, cnt)
    out = h_fin.reshape(B, T, NP, -1)[:, :, :N, :].transpose(0, 2, 1, 3)
    return out
